# baseline (speedup 1.0000x reference)
import jax
import jax.numpy as jnp
from jax.experimental import pallas as pl
from jax.experimental.pallas import tpu as pltpu

_K = 8
_NT = 2


def _body(x_ref, mid_ref, o_ref, ibuf, obuf, isem, osem):
    n = x_ref.shape[0]
    hw = x_ref.shape[-1]
    pos = jax.lax.broadcasted_iota(jnp.int32, (1, hw), 1)

    def in_copy(k):
        return pltpu.make_async_copy(x_ref.at[k], ibuf.at[k % _K], isem.at[k % _K])

    def out_copy(k):
        return pltpu.make_async_copy(obuf.at[k % _K], o_ref.at[k], osem.at[k % _K])

    for k in range(_K):
        in_copy(k).start(priority=k % _NT)
    for k in range(n):
        in_copy(k).wait()
        if k >= _K:
            out_copy(k - _K).wait()
        mid = mid_ref[k]
        obuf[k % _K] = jnp.where(pos == mid, 0.0, ibuf[k % _K])
        out_copy(k).start(priority=k % _NT)
        if k + _K < n:
            in_copy(k + _K).start(priority=(k + _K) % _NT)
    for k in range(n - _K, n):
        out_copy(k).wait()


def kernel(fea, mask_id):
    b, c, h, w = fea.shape
    hw = h * w
    x = fea.reshape(b, c, hw)
    out = pl.pallas_call(
        _body,
        grid=(1,),
        in_specs=[
            pl.BlockSpec(memory_space=pl.ANY),
            pl.BlockSpec(memory_space=pltpu.SMEM),
        ],
        out_specs=pl.BlockSpec(memory_space=pl.ANY),
        out_shape=jax.ShapeDtypeStruct((b, c, hw), jnp.float32),
        scratch_shapes=[
            pltpu.VMEM((_K, c, hw), jnp.float32),
            pltpu.VMEM((_K, c, hw), jnp.float32),
            pltpu.SemaphoreType.DMA((_K,)),
            pltpu.SemaphoreType.DMA((_K,)),
        ],
    )(x, mask_id)
    return out.reshape(b, c, h, w)
